# Initial kernel scaffold; baseline (speedup 1.0000x reference)
import jax
import jax.numpy as jnp
from jax.experimental import pallas as pl


def kernel(feat, gt):
    raise NotImplementedError("write your pallas kernel here")


# speedup vs baseline: 2.3757x; 2.3757x over previous
import functools

import jax
import jax.numpy as jnp
from jax.experimental import pallas as pl
from jax.experimental.pallas import tpu as pltpu

TAU = 0.07
EPS = 1e-12


def _loss_kernel(feat_ref, gt_ref, out_ref, k0_acc, cnt_acc, *, nsteps):
    step = pl.program_id(0)

    @pl.when(step == 0)
    def _init():
        k0_acc[...] = jnp.zeros_like(k0_acc)
        cnt_acc[...] = jnp.zeros_like(cnt_acc)

    c, hb, w_ = feat_ref.shape[1], feat_ref.shape[2], feat_ref.shape[3]
    k = gt_ref.shape[1]
    n = hb * w_

    feat = feat_ref[0].reshape(c, n)
    ss = jnp.sum(feat * feat, axis=0, keepdims=True)
    inv = 1.0 / jnp.maximum(jnp.sqrt(ss), EPS)

    pos = (gt_ref[0].reshape(k, n) == 1).astype(jnp.float32)
    cnt_acc[...] += jnp.sum(pos, axis=1, keepdims=True)
    posw = pos * inv
    k0_acc[...] += jax.lax.dot_general(
        posw, feat, (((1,), (1,)), ((), ())),
        preferred_element_type=jnp.float32)

    @pl.when(step == nsteps - 1)
    def _finalize():
        k0 = k0_acc[...]
        cnt = cnt_acc[...]
        k0_is = (cnt > 0.0).astype(jnp.float32)
        rown = jnp.sqrt(jnp.sum(k0 * k0, axis=1, keepdims=True))
        k0n = k0 / jnp.maximum(rown, EPS)
        logits = jax.lax.dot_general(
            k0n, k0n, (((1,), (1,)), ((), ())),
            preferred_element_type=jnp.float32) / TAU
        denom = jnp.sum(jnp.exp(logits), axis=0, keepdims=True)
        diag_logit = jnp.sum(k0n * k0n, axis=1, keepdims=True) / TAU
        terms = (jnp.log(denom).reshape(k, 1) - diag_logit) * k0_is
        out_ref[...] = (jnp.sum(terms) / jnp.sum(k0_is)).reshape(1, 1)


@jax.jit
def kernel(feat, gt):
    b, c, h, w = feat.shape
    k = gt.shape[1]
    hb = 16
    nsteps = b * (h // hb)

    out = pl.pallas_call(
        functools.partial(_loss_kernel, nsteps=nsteps),
        grid=(nsteps,),
        in_specs=[
            pl.BlockSpec((1, c, hb, w), lambda i: (i // (h // hb), 0, i % (h // hb), 0)),
            pl.BlockSpec((1, k, hb, w), lambda i: (i // (h // hb), 0, i % (h // hb), 0)),
        ],
        out_specs=pl.BlockSpec((1, 1), lambda i: (0, 0)),
        out_shape=jax.ShapeDtypeStruct((1, 1), jnp.float32),
        scratch_shapes=[
            pltpu.VMEM((k, c), jnp.float32),
            pltpu.VMEM((k, 1), jnp.float32),
        ],
    )(feat, gt)
    return out.reshape(1)


# speedup vs baseline: 2.3890x; 1.0056x over previous
import functools

import jax
import jax.numpy as jnp
from jax.experimental import pallas as pl
from jax.experimental.pallas import tpu as pltpu

TAU = 0.07
EPS = 1e-12


def _loss_kernel(feat_ref, gt_ref, out_ref, k0_acc, cnt_acc, *, nsteps):
    step = pl.program_id(0)

    @pl.when(step == 0)
    def _init():
        k0_acc[...] = jnp.zeros_like(k0_acc)
        cnt_acc[...] = jnp.zeros_like(cnt_acc)

    c, hb, w_ = feat_ref.shape[1], feat_ref.shape[2], feat_ref.shape[3]
    k = gt_ref.shape[1]
    n = hb * w_

    feat = feat_ref[0].reshape(c, n)
    fsq = feat * feat
    ss = jax.lax.dot_general(
        jnp.ones((1, c), jnp.float32), fsq, (((1,), (0,)), ((), ())),
        preferred_element_type=jnp.float32)
    inv = 1.0 / jnp.maximum(jnp.sqrt(ss), EPS)

    pos = gt_ref[0].reshape(k, n).astype(jnp.float32)
    cnt_acc[...] += jax.lax.dot_general(
        pos, jnp.ones((1, n), jnp.float32), (((1,), (1,)), ((), ())),
        preferred_element_type=jnp.float32)
    posw = pos * inv
    k0_acc[...] += jax.lax.dot_general(
        posw, feat, (((1,), (1,)), ((), ())),
        preferred_element_type=jnp.float32)

    @pl.when(step == nsteps - 1)
    def _finalize():
        k0 = k0_acc[...]
        cnt = cnt_acc[...]
        k0_is = (cnt > 0.0).astype(jnp.float32)
        rown = jnp.sqrt(jnp.sum(k0 * k0, axis=1, keepdims=True))
        k0n = k0 / jnp.maximum(rown, EPS)
        logits = jax.lax.dot_general(
            k0n, k0n, (((1,), (1,)), ((), ())),
            preferred_element_type=jnp.float32) / TAU
        denom = jnp.sum(jnp.exp(logits), axis=0, keepdims=True)
        diag_logit = jnp.sum(k0n * k0n, axis=1, keepdims=True) / TAU
        terms = (jnp.log(denom).reshape(k, 1) - diag_logit) * k0_is
        out_ref[...] = (jnp.sum(terms) / jnp.sum(k0_is)).reshape(1, 1)


@jax.jit
def kernel(feat, gt):
    b, c, h, w = feat.shape
    k = gt.shape[1]
    hb = 16
    nsteps = b * (h // hb)

    out = pl.pallas_call(
        functools.partial(_loss_kernel, nsteps=nsteps),
        grid=(nsteps,),
        in_specs=[
            pl.BlockSpec((1, c, hb, w), lambda i: (i // (h // hb), 0, i % (h // hb), 0)),
            pl.BlockSpec((1, k, hb, w), lambda i: (i // (h // hb), 0, i % (h // hb), 0)),
        ],
        out_specs=pl.BlockSpec((1, 1), lambda i: (0, 0)),
        out_shape=jax.ShapeDtypeStruct((1, 1), jnp.float32),
        scratch_shapes=[
            pltpu.VMEM((k, c), jnp.float32),
            pltpu.VMEM((k, 1), jnp.float32),
        ],
    )(feat, gt)
    return out.reshape(1)
